# Initial kernel scaffold; baseline (speedup 1.0000x reference)
import jax
import jax.numpy as jnp
from jax.experimental import pallas as pl


def kernel(previous_ast_nodes_encodings, new_cfg_nodes_encodings, key_indices, value_indices, W_update, b_update, W_gate, b_gate):
    raise NotImplementedError("write your pallas kernel here")


# speedup vs baseline: 3.9651x; 3.9651x over previous
import functools

import jax
import jax.numpy as jnp
from jax import lax
from jax.experimental import pallas as pl
from jax.experimental.pallas import tpu as pltpu
from jax.experimental.pallas import tpu_sc as plsc

_NC = 2
_NS = 16
_NW = _NC * _NS


def _sc_row_gather(vsel, cfg, n_pad, d, chunk):
  rows_per_w = n_pad // _NW
  n_chunks = rows_per_w // chunk
  mesh = plsc.VectorSubcoreMesh(core_axis_name="c", subcore_axis_name="s")

  @functools.partial(
      pl.kernel,
      out_type=jax.ShapeDtypeStruct((n_pad, d), jnp.float32),
      mesh=mesh,
      scratch_types=[
          pltpu.VMEM((rows_per_w,), jnp.int32),
          pltpu.VMEM((chunk, d), jnp.float32),
          pltpu.VMEM((chunk, d), jnp.float32),
          pltpu.SemaphoreType.DMA,
          pltpu.SemaphoreType.DMA,
      ],
  )
  def k(vsel_hbm, cfg_hbm, out_hbm, idx_v, buf0, buf1, sem0, sem1):
    wid = lax.axis_index("s") * _NC + lax.axis_index("c")
    base = wid * rows_per_w
    pltpu.sync_copy(vsel_hbm.at[pl.ds(base, rows_per_w)], idx_v)

    def gather_chunk(c, buf, sem):
      pltpu.async_copy(cfg_hbm.at[idx_v.at[pl.ds(c * chunk, chunk)]], buf, sem)

    gather_chunk(0, buf0, sem0)
    for c in range(n_chunks):
      cur, csem = (buf0, sem0) if c % 2 == 0 else (buf1, sem1)
      nxt, nsem = (buf1, sem1) if c % 2 == 0 else (buf0, sem0)
      if c + 1 < n_chunks:
        gather_chunk(c + 1, nxt, nsem)
      pltpu.make_async_copy(
          cfg_hbm.at[idx_v.at[pl.ds(c * chunk, chunk)]], cur, csem).wait()
      pltpu.sync_copy(cur, out_hbm.at[pl.ds(base + c * chunk, chunk)])

  return k(vsel, cfg)


def _tc_mlp_body(prev_ref, upd_ref, wp_ref, wu_ref, wg1_ref, wg2_ref,
                 bu_ref, bg_ref, out_ref):
  prev = prev_ref[...]
  upd = upd_ref[...]
  proj = jnp.maximum(
      jnp.dot(upd, wu_ref[...], preferred_element_type=jnp.float32)
      + bu_ref[...], 0.0)
  z = (jnp.dot(prev, wg1_ref[...], preferred_element_type=jnp.float32)
       + jnp.dot(proj, wg2_ref[...], preferred_element_type=jnp.float32)
       + bg_ref[...])
  gate = jax.nn.sigmoid(z)
  newr = gate * prev + (1.0 - gate) * proj
  out_ref[...] = jnp.where(wp_ref[...] >= 0, newr, prev)


def _tc_mlp(prev_table, upd_sel, winpos2d, wu, wg1, wg2, bu, bg, blk):
  n, d = prev_table.shape
  grid = (n // blk,)
  return pl.pallas_call(
      _tc_mlp_body,
      grid=grid,
      in_specs=[
          pl.BlockSpec((blk, d), lambda i: (i, 0)),
          pl.BlockSpec((blk, d), lambda i: (i, 0)),
          pl.BlockSpec((blk, 1), lambda i: (i, 0)),
          pl.BlockSpec((d, d), lambda i: (0, 0)),
          pl.BlockSpec((d, d), lambda i: (0, 0)),
          pl.BlockSpec((d, d), lambda i: (0, 0)),
          pl.BlockSpec((1, d), lambda i: (0, 0)),
          pl.BlockSpec((1, d), lambda i: (0, 0)),
      ],
      out_specs=pl.BlockSpec((blk, d), lambda i: (i, 0)),
      out_shape=jax.ShapeDtypeStruct((n, d), jnp.float32),
  )(prev_table, upd_sel, winpos2d, wu, wg1, wg2, bu, bg)


def kernel(previous_ast_nodes_encodings, new_cfg_nodes_encodings, key_indices,
           value_indices, W_update, b_update, W_gate, b_gate):
  n_ast, d = previous_ast_nodes_encodings.shape
  n_cfg = new_cfg_nodes_encodings.shape[0]
  e = key_indices.shape[0]

  key_indices = key_indices.astype(jnp.int32)
  value_indices = value_indices.astype(jnp.int32)

  positions = jnp.arange(e, dtype=jnp.int32)
  winpos = jnp.full((n_ast,), -1, jnp.int32).at[key_indices].max(positions)
  vsel = value_indices[jnp.maximum(winpos, 0)]
  row_ids = jnp.arange(n_ast, dtype=jnp.int32)
  vsel = jnp.where(winpos >= 0, vsel, row_ids % n_cfg)

  n_pad = 102400
  pad = n_pad - n_ast
  vsel_p = jnp.concatenate(
      [vsel, (jnp.arange(pad, dtype=jnp.int32) % n_cfg)])

  upd_sel = _sc_row_gather(vsel_p, new_cfg_nodes_encodings, n_pad, d,
                           chunk=160)

  wg1 = W_gate[:d]
  wg2 = W_gate[d:]
  bu = b_update.reshape(1, d)
  bg = b_gate.reshape(1, d)
  winpos2d = winpos.reshape(n_ast, 1)

  out = _tc_mlp(previous_ast_nodes_encodings, upd_sel, winpos2d,
                W_update, wg1, wg2, bu, bg, blk=400)
  return out


# speedup vs baseline: 7.3794x; 1.8611x over previous
import functools

import jax
import jax.numpy as jnp
from jax import lax
from jax.experimental import pallas as pl
from jax.experimental.pallas import tpu as pltpu
from jax.experimental.pallas import tpu_sc as plsc

_NC = 2
_NS = 16
_NW = _NC * _NS


def _sc_winpos_phase1(keys_pad, n_rows_pad, e_pad):
  per_w = e_pad // _NW
  nvec = per_w // 16
  mesh = plsc.VectorSubcoreMesh(core_axis_name="c", subcore_axis_name="s")

  @functools.partial(
      pl.kernel,
      out_type=jax.ShapeDtypeStruct((_NW, n_rows_pad), jnp.int32),
      mesh=mesh,
      compiler_params=pltpu.CompilerParams(needs_layout_passes=False),
      scratch_types=[
          pltpu.VMEM((per_w,), jnp.int32),
          pltpu.VMEM((n_rows_pad,), jnp.int32),
      ],
  )
  def k(keys_hbm, out_hbm, keys_v, lwin):
    wid = lax.axis_index("s") * _NC + lax.axis_index("c")
    base = wid * per_w
    pltpu.sync_copy(keys_hbm.at[pl.ds(base, per_w)], keys_v)

    minus1 = jnp.full((16,), -1, jnp.int32)

    def init_body(i, carry):
      for j in range(8):
        lwin[pl.ds((i * 8 + j) * 16, 16)] = minus1
      return carry

    lax.fori_loop(0, n_rows_pad // (16 * 8), init_body, 0)

    lane = lax.iota(jnp.int32, 16)

    def scat_body(i, carry):
      keys = keys_v[pl.ds(i * 16, 16)]
      pos = base + i * 16 + lane
      for l in range(16):
        plsc.store_scatter(lwin, [keys], pos, mask=lane == l)
      return carry

    lax.fori_loop(0, nvec, scat_body, 0)
    pltpu.sync_copy(lwin, out_hbm.at[wid])

  return k(keys_pad)


def _sc_winpos_phase2(lwin_all, value_idx_pad, n_rows_pad, n_cfg):
  per_w = n_rows_pad // _NW
  nv = per_w // 16
  mesh = plsc.VectorSubcoreMesh(core_axis_name="c", subcore_axis_name="s")

  @functools.partial(
      pl.kernel,
      out_type=(jax.ShapeDtypeStruct((n_rows_pad,), jnp.int32),
                jax.ShapeDtypeStruct((n_rows_pad,), jnp.int32)),
      mesh=mesh,
      scratch_types=[
          pltpu.VMEM((_NW, per_w), jnp.int32),
          pltpu.VMEM((per_w,), jnp.int32),
          pltpu.VMEM((per_w,), jnp.int32),
          pltpu.VMEM((per_w,), jnp.int32),
          pltpu.SemaphoreType.DMA,
          pltpu.SemaphoreType.DMA,
      ],
  )
  def k(lwin_hbm, vi_hbm, wp_hbm, vs_hbm, tbl_v, wp_v, clamp_v, vs_v,
        sem, sem2):
    wid = lax.axis_index("s") * _NC + lax.axis_index("c")
    base = wid * per_w
    for t in range(_NW):
      pltpu.async_copy(lwin_hbm.at[t, pl.ds(base, per_w)], tbl_v.at[t], sem)
    for t in range(_NW):
      pltpu.make_async_copy(
          lwin_hbm.at[t, pl.ds(base, per_w)], tbl_v.at[t], sem).wait()

    def max_body(i, carry):
      acc = tbl_v[0, pl.ds(i * 16, 16)]
      for t in range(1, _NW):
        acc = jnp.maximum(acc, tbl_v[t, pl.ds(i * 16, 16)])
      wp_v[pl.ds(i * 16, 16)] = acc
      clamp_v[pl.ds(i * 16, 16)] = jnp.maximum(acc, 0)
      return carry

    lax.fori_loop(0, nv, max_body, 0)

    pltpu.async_copy(vi_hbm.at[clamp_v], vs_v, sem2).wait()

    lane = lax.iota(jnp.int32, 16)

    def sel_body(i, carry):
      wp = wp_v[pl.ds(i * 16, 16)]
      g = vs_v[pl.ds(i * 16, 16)]
      rows = base + i * 16 + lane
      dummy = lax.rem(rows, n_cfg)
      vs_v[pl.ds(i * 16, 16)] = jnp.where(wp >= 0, g, dummy)
      return carry

    lax.fori_loop(0, nv, sel_body, 0)
    pltpu.sync_copy(wp_v, wp_hbm.at[pl.ds(base, per_w)])
    pltpu.sync_copy(vs_v, vs_hbm.at[pl.ds(base, per_w)])

  return k(lwin_all, value_idx_pad)


def _sc_row_gather(vsel, cfg, n_pad, d, chunk):
  rows_per_w = n_pad // _NW
  n_chunks = rows_per_w // chunk
  mesh = plsc.VectorSubcoreMesh(core_axis_name="c", subcore_axis_name="s")

  @functools.partial(
      pl.kernel,
      out_type=jax.ShapeDtypeStruct((n_pad, d), jnp.float32),
      mesh=mesh,
      scratch_types=[
          pltpu.VMEM((rows_per_w,), jnp.int32),
          pltpu.VMEM((chunk, d), jnp.float32),
          pltpu.VMEM((chunk, d), jnp.float32),
          pltpu.SemaphoreType.DMA,
          pltpu.SemaphoreType.DMA,
      ],
  )
  def k(vsel_hbm, cfg_hbm, out_hbm, idx_v, buf0, buf1, sem0, sem1):
    wid = lax.axis_index("s") * _NC + lax.axis_index("c")
    base = wid * rows_per_w
    pltpu.sync_copy(vsel_hbm.at[pl.ds(base, rows_per_w)], idx_v)

    def gather_chunk(c, buf, sem):
      pltpu.async_copy(cfg_hbm.at[idx_v.at[pl.ds(c * chunk, chunk)]], buf, sem)

    gather_chunk(0, buf0, sem0)
    for c in range(n_chunks):
      cur, csem = (buf0, sem0) if c % 2 == 0 else (buf1, sem1)
      nxt, nsem = (buf1, sem1) if c % 2 == 0 else (buf0, sem0)
      if c + 1 < n_chunks:
        gather_chunk(c + 1, nxt, nsem)
      pltpu.make_async_copy(
          cfg_hbm.at[idx_v.at[pl.ds(c * chunk, chunk)]], cur, csem).wait()
      pltpu.sync_copy(cur, out_hbm.at[pl.ds(base + c * chunk, chunk)])

  return k(vsel, cfg)


def _tc_mlp_body(prev_ref, upd_ref, wp_ref, wu_ref, wg1_ref, wg2_ref,
                 bu_ref, bg_ref, out_ref):
  prev = prev_ref[...]
  upd = upd_ref[...]
  proj = jnp.maximum(
      jnp.dot(upd, wu_ref[...], preferred_element_type=jnp.float32)
      + bu_ref[...], 0.0)
  z = (jnp.dot(prev, wg1_ref[...], preferred_element_type=jnp.float32)
       + jnp.dot(proj, wg2_ref[...], preferred_element_type=jnp.float32)
       + bg_ref[...])
  gate = jax.nn.sigmoid(z)
  newr = gate * prev + (1.0 - gate) * proj
  out_ref[...] = jnp.where(wp_ref[...] >= 0, newr, prev)


def _tc_mlp(prev_table, upd_sel, winpos2d, wu, wg1, wg2, bu, bg, blk):
  n, d = prev_table.shape
  grid = (n // blk,)
  return pl.pallas_call(
      _tc_mlp_body,
      grid=grid,
      in_specs=[
          pl.BlockSpec((blk, d), lambda i: (i, 0)),
          pl.BlockSpec((blk, d), lambda i: (i, 0)),
          pl.BlockSpec((blk, 1), lambda i: (i, 0)),
          pl.BlockSpec((d, d), lambda i: (0, 0)),
          pl.BlockSpec((d, d), lambda i: (0, 0)),
          pl.BlockSpec((d, d), lambda i: (0, 0)),
          pl.BlockSpec((1, d), lambda i: (0, 0)),
          pl.BlockSpec((1, d), lambda i: (0, 0)),
      ],
      out_specs=pl.BlockSpec((blk, d), lambda i: (i, 0)),
      out_shape=jax.ShapeDtypeStruct((n, d), jnp.float32),
  )(prev_table, upd_sel, winpos2d, wu, wg1, wg2, bu, bg)


def kernel(previous_ast_nodes_encodings, new_cfg_nodes_encodings, key_indices,
           value_indices, W_update, b_update, W_gate, b_gate):
  n_ast, d = previous_ast_nodes_encodings.shape
  n_cfg = new_cfg_nodes_encodings.shape[0]
  e = key_indices.shape[0]

  key_indices = key_indices.astype(jnp.int32)
  value_indices = value_indices.astype(jnp.int32)

  n_pad = 102400
  e_pad = 200704

  pad_e = e_pad - e
  keys_pad = jnp.concatenate(
      [key_indices, n_ast + (jnp.arange(pad_e, dtype=jnp.int32) % 96)])
  vi_pad = jnp.concatenate(
      [value_indices, jnp.arange(pad_e, dtype=jnp.int32) % n_cfg])

  lwin_all = _sc_winpos_phase1(keys_pad, n_pad, e_pad)
  winpos_p, vsel_p = _sc_winpos_phase2(lwin_all, vi_pad, n_pad, n_cfg)

  upd_sel = _sc_row_gather(vsel_p, new_cfg_nodes_encodings, n_pad, d,
                           chunk=160)

  wg1 = W_gate[:d]
  wg2 = W_gate[d:]
  bu = b_update.reshape(1, d)
  bg = b_gate.reshape(1, d)
  winpos2d = winpos_p.reshape(n_pad, 1)

  out = _tc_mlp(previous_ast_nodes_encodings, upd_sel, winpos2d,
                W_update, wg1, wg2, bu, bg, blk=400)
  return out


# speedup vs baseline: 7.3819x; 1.0003x over previous
import functools

import jax
import jax.numpy as jnp
from jax import lax
from jax.experimental import pallas as pl
from jax.experimental.pallas import tpu as pltpu
from jax.experimental.pallas import tpu_sc as plsc

_NC = 2
_NS = 16
_NW = _NC * _NS


def _sc_winpos_phase1(keys_pad, n_rows_pad, e_pad):
  per_w = e_pad // _NW
  nvec = per_w // 16
  mesh = plsc.VectorSubcoreMesh(core_axis_name="c", subcore_axis_name="s")

  @functools.partial(
      pl.kernel,
      out_type=jax.ShapeDtypeStruct((_NW, n_rows_pad), jnp.int32),
      mesh=mesh,
      compiler_params=pltpu.CompilerParams(needs_layout_passes=False),
      scratch_types=[
          pltpu.VMEM((per_w,), jnp.int32),
          pltpu.VMEM((n_rows_pad,), jnp.int32),
      ],
  )
  def k(keys_hbm, out_hbm, keys_v, lwin):
    wid = lax.axis_index("s") * _NC + lax.axis_index("c")
    base = wid * per_w
    pltpu.sync_copy(keys_hbm.at[pl.ds(base, per_w)], keys_v)

    minus1 = jnp.full((16,), -1, jnp.int32)

    def init_body(i, carry):
      for j in range(8):
        lwin[pl.ds((i * 8 + j) * 16, 16)] = minus1
      return carry

    lax.fori_loop(0, n_rows_pad // (16 * 8), init_body, 0)

    lane = lax.iota(jnp.int32, 16)

    def scat_body(i, carry):
      keys = keys_v[pl.ds(i * 16, 16)]
      pos = base + i * 16 + lane
      for l in range(16):
        plsc.store_scatter(lwin, [keys], pos, mask=lane == l)
      return carry

    lax.fori_loop(0, nvec, scat_body, 0)
    pltpu.sync_copy(lwin, out_hbm.at[wid])

  return k(keys_pad)


def _sc_winpos_phase2(lwin_all, value_idx_pad, n_rows_pad, n_cfg):
  per_w = n_rows_pad // _NW
  nv = per_w // 16
  mesh = plsc.VectorSubcoreMesh(core_axis_name="c", subcore_axis_name="s")

  @functools.partial(
      pl.kernel,
      out_type=(jax.ShapeDtypeStruct((n_rows_pad,), jnp.int32),
                jax.ShapeDtypeStruct((n_rows_pad,), jnp.int32)),
      mesh=mesh,
      scratch_types=[
          pltpu.VMEM((_NW, per_w), jnp.int32),
          pltpu.VMEM((per_w,), jnp.int32),
          pltpu.VMEM((per_w,), jnp.int32),
          pltpu.VMEM((per_w,), jnp.int32),
          pltpu.SemaphoreType.DMA,
          pltpu.SemaphoreType.DMA,
      ],
  )
  def k(lwin_hbm, vi_hbm, wp_hbm, vs_hbm, tbl_v, wp_v, clamp_v, vs_v,
        sem, sem2):
    wid = lax.axis_index("s") * _NC + lax.axis_index("c")
    base = wid * per_w
    for t in range(_NW):
      pltpu.async_copy(lwin_hbm.at[t, pl.ds(base, per_w)], tbl_v.at[t], sem)
    for t in range(_NW):
      pltpu.make_async_copy(
          lwin_hbm.at[t, pl.ds(base, per_w)], tbl_v.at[t], sem).wait()

    def max_body(i, carry):
      acc = tbl_v[0, pl.ds(i * 16, 16)]
      for t in range(1, _NW):
        acc = jnp.maximum(acc, tbl_v[t, pl.ds(i * 16, 16)])
      wp_v[pl.ds(i * 16, 16)] = acc
      clamp_v[pl.ds(i * 16, 16)] = jnp.maximum(acc, 0)
      return carry

    lax.fori_loop(0, nv, max_body, 0)

    pltpu.async_copy(vi_hbm.at[clamp_v], vs_v, sem2).wait()

    lane = lax.iota(jnp.int32, 16)

    dummy_mask = 32767
    assert n_cfg > dummy_mask

    def sel_body(i, carry):
      wp = wp_v[pl.ds(i * 16, 16)]
      g = vs_v[pl.ds(i * 16, 16)]
      rows = base + i * 16 + lane
      dummy = rows & dummy_mask
      vs_v[pl.ds(i * 16, 16)] = jnp.where(wp >= 0, g, dummy)
      return carry

    lax.fori_loop(0, nv, sel_body, 0)
    pltpu.sync_copy(wp_v, wp_hbm.at[pl.ds(base, per_w)])
    pltpu.sync_copy(vs_v, vs_hbm.at[pl.ds(base, per_w)])

  return k(lwin_all, value_idx_pad)


def _sc_row_gather(vsel, cfg, n_pad, d, chunk):
  rows_per_w = n_pad // _NW
  n_chunks = rows_per_w // chunk
  mesh = plsc.VectorSubcoreMesh(core_axis_name="c", subcore_axis_name="s")

  @functools.partial(
      pl.kernel,
      out_type=jax.ShapeDtypeStruct((n_pad, d), jnp.float32),
      mesh=mesh,
      scratch_types=[
          pltpu.VMEM((rows_per_w,), jnp.int32),
          pltpu.VMEM((chunk, d), jnp.float32),
          pltpu.VMEM((chunk, d), jnp.float32),
          pltpu.SemaphoreType.DMA,
          pltpu.SemaphoreType.DMA,
      ],
  )
  def k(vsel_hbm, cfg_hbm, out_hbm, idx_v, buf0, buf1, sem0, sem1):
    wid = lax.axis_index("s") * _NC + lax.axis_index("c")
    base = wid * rows_per_w
    pltpu.sync_copy(vsel_hbm.at[pl.ds(base, rows_per_w)], idx_v)

    def gather_chunk(c, buf, sem):
      pltpu.async_copy(cfg_hbm.at[idx_v.at[pl.ds(c * chunk, chunk)]], buf, sem)

    gather_chunk(0, buf0, sem0)
    for c in range(n_chunks):
      cur, csem = (buf0, sem0) if c % 2 == 0 else (buf1, sem1)
      nxt, nsem = (buf1, sem1) if c % 2 == 0 else (buf0, sem0)
      if c + 1 < n_chunks:
        gather_chunk(c + 1, nxt, nsem)
      pltpu.make_async_copy(
          cfg_hbm.at[idx_v.at[pl.ds(c * chunk, chunk)]], cur, csem).wait()
      pltpu.sync_copy(cur, out_hbm.at[pl.ds(base + c * chunk, chunk)])

  return k(vsel, cfg)


def _tc_mlp_body(prev_ref, upd_ref, wp_ref, wu_ref, wg1_ref, wg2_ref,
                 bu_ref, bg_ref, out_ref):
  prev = prev_ref[...]
  upd = upd_ref[...]
  proj = jnp.maximum(
      jnp.dot(upd, wu_ref[...], preferred_element_type=jnp.float32)
      + bu_ref[...], 0.0)
  z = (jnp.dot(prev, wg1_ref[...], preferred_element_type=jnp.float32)
       + jnp.dot(proj, wg2_ref[...], preferred_element_type=jnp.float32)
       + bg_ref[...])
  gate = jax.nn.sigmoid(z)
  newr = gate * prev + (1.0 - gate) * proj
  out_ref[...] = jnp.where(wp_ref[...] >= 0, newr, prev)


def _tc_mlp(prev_table, upd_sel, winpos2d, wu, wg1, wg2, bu, bg, blk):
  n, d = prev_table.shape
  grid = (n // blk,)
  return pl.pallas_call(
      _tc_mlp_body,
      grid=grid,
      in_specs=[
          pl.BlockSpec((blk, d), lambda i: (i, 0)),
          pl.BlockSpec((blk, d), lambda i: (i, 0)),
          pl.BlockSpec((blk, 1), lambda i: (i, 0)),
          pl.BlockSpec((d, d), lambda i: (0, 0)),
          pl.BlockSpec((d, d), lambda i: (0, 0)),
          pl.BlockSpec((d, d), lambda i: (0, 0)),
          pl.BlockSpec((1, d), lambda i: (0, 0)),
          pl.BlockSpec((1, d), lambda i: (0, 0)),
      ],
      out_specs=pl.BlockSpec((blk, d), lambda i: (i, 0)),
      out_shape=jax.ShapeDtypeStruct((n, d), jnp.float32),
  )(prev_table, upd_sel, winpos2d, wu, wg1, wg2, bu, bg)


def kernel(previous_ast_nodes_encodings, new_cfg_nodes_encodings, key_indices,
           value_indices, W_update, b_update, W_gate, b_gate):
  n_ast, d = previous_ast_nodes_encodings.shape
  n_cfg = new_cfg_nodes_encodings.shape[0]
  e = key_indices.shape[0]

  key_indices = key_indices.astype(jnp.int32)
  value_indices = value_indices.astype(jnp.int32)

  n_pad = 102400
  e_pad = 200704

  pad_e = e_pad - e
  keys_pad = jnp.concatenate(
      [key_indices, n_ast + (jnp.arange(pad_e, dtype=jnp.int32) % 96)])
  vi_pad = jnp.concatenate(
      [value_indices, jnp.arange(pad_e, dtype=jnp.int32) % n_cfg])

  lwin_all = _sc_winpos_phase1(keys_pad, n_pad, e_pad)
  winpos_p, vsel_p = _sc_winpos_phase2(lwin_all, vi_pad, n_pad, n_cfg)

  upd_sel = _sc_row_gather(vsel_p, new_cfg_nodes_encodings, n_pad, d,
                           chunk=160)

  wg1 = W_gate[:d]
  wg2 = W_gate[d:]
  bu = b_update.reshape(1, d)
  bg = b_gate.reshape(1, d)
  winpos2d = winpos_p.reshape(n_pad, 1)

  out = _tc_mlp(previous_ast_nodes_encodings, upd_sel, winpos2d,
                W_update, wg1, wg2, bu, bg, blk=400)
  return out


# speedup vs baseline: 8.7162x; 1.1808x over previous
import functools

import jax
import jax.numpy as jnp
from jax import lax
from jax.experimental import pallas as pl
from jax.experimental.pallas import tpu as pltpu
from jax.experimental.pallas import tpu_sc as plsc

_NC = 2
_NS = 16
_NW = _NC * _NS


def _sc_winpos_phase1(keys_pad, n_rows_pad, e_pad):
  per_w = e_pad // _NW
  nvec = per_w // 16
  mesh = plsc.VectorSubcoreMesh(core_axis_name="c", subcore_axis_name="s")

  slice_w = n_rows_pad // _NW

  @functools.partial(
      pl.kernel,
      out_type=jax.ShapeDtypeStruct((_NW, _NW, slice_w), jnp.int32),
      mesh=mesh,
      compiler_params=pltpu.CompilerParams(needs_layout_passes=False),
      scratch_types=[
          pltpu.VMEM((per_w,), jnp.int32),
          pltpu.VMEM((n_rows_pad,), jnp.int32),
          pltpu.SemaphoreType.DMA,
      ],
  )
  def k(keys_hbm, out_hbm, keys_v, lwin, osem):
    wid = lax.axis_index("s") * _NC + lax.axis_index("c")
    base = wid * per_w
    pltpu.sync_copy(keys_hbm.at[pl.ds(base, per_w)], keys_v)

    minus1 = jnp.full((16,), -1, jnp.int32)

    def init_body(i, carry):
      for j in range(8):
        lwin[pl.ds((i * 8 + j) * 16, 16)] = minus1
      return carry

    lax.fori_loop(0, n_rows_pad // (16 * 8), init_body, 0)

    lane = lax.iota(jnp.int32, 16)

    def scat_body(i, carry):
      keys = keys_v[pl.ds(i * 16, 16)]
      pos = base + i * 16 + lane
      for l in range(16):
        plsc.store_scatter(lwin, [keys], pos, mask=lane == l)
      return carry

    lax.fori_loop(0, nvec, scat_body, 0)
    for s in range(_NW):
      pltpu.async_copy(lwin.at[pl.ds(s * slice_w, slice_w)],
                       out_hbm.at[s, wid], osem)
    for s in range(_NW):
      pltpu.make_async_copy(lwin.at[pl.ds(s * slice_w, slice_w)],
                            out_hbm.at[s, wid], osem).wait()

  return k(keys_pad)


def _sc_winpos_phase2(lwin_all, value_idx_pad, n_rows_pad, n_cfg):
  per_w = n_rows_pad // _NW
  nv = per_w // 16
  mesh = plsc.VectorSubcoreMesh(core_axis_name="c", subcore_axis_name="s")

  @functools.partial(
      pl.kernel,
      out_type=(jax.ShapeDtypeStruct((n_rows_pad,), jnp.int32),
                jax.ShapeDtypeStruct((n_rows_pad,), jnp.int32)),
      mesh=mesh,
      scratch_types=[
          pltpu.VMEM((_NW, per_w), jnp.int32),
          pltpu.VMEM((per_w,), jnp.int32),
          pltpu.VMEM((per_w,), jnp.int32),
          pltpu.VMEM((per_w,), jnp.int32),
          pltpu.SemaphoreType.DMA,
          pltpu.SemaphoreType.DMA,
      ],
  )
  def k(lwin_hbm, vi_hbm, wp_hbm, vs_hbm, tbl_v, wp_v, clamp_v, vs_v,
        sem, sem2):
    wid = lax.axis_index("s") * _NC + lax.axis_index("c")
    base = wid * per_w
    pltpu.sync_copy(lwin_hbm.at[wid], tbl_v)

    def max_body(i, carry):
      acc = tbl_v[0, pl.ds(i * 16, 16)]
      for t in range(1, _NW):
        acc = jnp.maximum(acc, tbl_v[t, pl.ds(i * 16, 16)])
      wp_v[pl.ds(i * 16, 16)] = acc
      clamp_v[pl.ds(i * 16, 16)] = jnp.maximum(acc, 0)
      return carry

    lax.fori_loop(0, nv, max_body, 0)

    pltpu.async_copy(vi_hbm.at[clamp_v], vs_v, sem2).wait()

    lane = lax.iota(jnp.int32, 16)

    dummy_mask = 32767
    assert n_cfg > dummy_mask

    def sel_body(i, carry):
      wp = wp_v[pl.ds(i * 16, 16)]
      g = vs_v[pl.ds(i * 16, 16)]
      rows = base + i * 16 + lane
      dummy = rows & dummy_mask
      vs_v[pl.ds(i * 16, 16)] = jnp.where(wp >= 0, g, dummy)
      return carry

    lax.fori_loop(0, nv, sel_body, 0)
    pltpu.sync_copy(wp_v, wp_hbm.at[pl.ds(base, per_w)])
    pltpu.sync_copy(vs_v, vs_hbm.at[pl.ds(base, per_w)])

  return k(lwin_all, value_idx_pad)


def _sc_row_gather(vsel, cfg, n_pad, d, chunk):
  rows_per_w = n_pad // _NW
  n_chunks = rows_per_w // chunk
  mesh = plsc.VectorSubcoreMesh(core_axis_name="c", subcore_axis_name="s")

  @functools.partial(
      pl.kernel,
      out_type=jax.ShapeDtypeStruct((n_pad, d), jnp.float32),
      mesh=mesh,
      scratch_types=[
          pltpu.VMEM((rows_per_w,), jnp.int32),
          pltpu.VMEM((chunk, d), jnp.float32),
          pltpu.VMEM((chunk, d), jnp.float32),
          pltpu.SemaphoreType.DMA,
          pltpu.SemaphoreType.DMA,
      ],
  )
  def k(vsel_hbm, cfg_hbm, out_hbm, idx_v, buf0, buf1, sem0, sem1):
    wid = lax.axis_index("s") * _NC + lax.axis_index("c")
    base = wid * rows_per_w
    pltpu.sync_copy(vsel_hbm.at[pl.ds(base, rows_per_w)], idx_v)

    def gather_chunk(c, buf, sem):
      pltpu.async_copy(cfg_hbm.at[idx_v.at[pl.ds(c * chunk, chunk)]], buf, sem)

    gather_chunk(0, buf0, sem0)
    for c in range(n_chunks):
      cur, csem = (buf0, sem0) if c % 2 == 0 else (buf1, sem1)
      nxt, nsem = (buf1, sem1) if c % 2 == 0 else (buf0, sem0)
      if c + 1 < n_chunks:
        gather_chunk(c + 1, nxt, nsem)
      pltpu.make_async_copy(
          cfg_hbm.at[idx_v.at[pl.ds(c * chunk, chunk)]], cur, csem).wait()
      pltpu.sync_copy(cur, out_hbm.at[pl.ds(base + c * chunk, chunk)])

  return k(vsel, cfg)


def _tc_mlp_body(prev_ref, upd_ref, wp_ref, wu_ref, wg1_ref, wg2_ref,
                 bu_ref, bg_ref, out_ref):
  prev = prev_ref[...]
  upd = upd_ref[...]
  proj = jnp.maximum(
      jnp.dot(upd, wu_ref[...], preferred_element_type=jnp.float32)
      + bu_ref[...], 0.0)
  z = (jnp.dot(prev, wg1_ref[...], preferred_element_type=jnp.float32)
       + jnp.dot(proj, wg2_ref[...], preferred_element_type=jnp.float32)
       + bg_ref[...])
  gate = jax.nn.sigmoid(z)
  newr = gate * prev + (1.0 - gate) * proj
  out_ref[...] = jnp.where(wp_ref[...] >= 0, newr, prev)


def _tc_mlp(prev_table, upd_sel, winpos2d, wu, wg1, wg2, bu, bg, blk):
  n, d = prev_table.shape
  grid = (n // blk,)
  return pl.pallas_call(
      _tc_mlp_body,
      grid=grid,
      in_specs=[
          pl.BlockSpec((blk, d), lambda i: (i, 0)),
          pl.BlockSpec((blk, d), lambda i: (i, 0)),
          pl.BlockSpec((blk, 1), lambda i: (i, 0)),
          pl.BlockSpec((d, d), lambda i: (0, 0)),
          pl.BlockSpec((d, d), lambda i: (0, 0)),
          pl.BlockSpec((d, d), lambda i: (0, 0)),
          pl.BlockSpec((1, d), lambda i: (0, 0)),
          pl.BlockSpec((1, d), lambda i: (0, 0)),
      ],
      out_specs=pl.BlockSpec((blk, d), lambda i: (i, 0)),
      out_shape=jax.ShapeDtypeStruct((n, d), jnp.float32),
  )(prev_table, upd_sel, winpos2d, wu, wg1, wg2, bu, bg)


def kernel(previous_ast_nodes_encodings, new_cfg_nodes_encodings, key_indices,
           value_indices, W_update, b_update, W_gate, b_gate):
  n_ast, d = previous_ast_nodes_encodings.shape
  n_cfg = new_cfg_nodes_encodings.shape[0]
  e = key_indices.shape[0]

  key_indices = key_indices.astype(jnp.int32)
  value_indices = value_indices.astype(jnp.int32)

  n_pad = 102400
  e_pad = 200704

  pad_e = e_pad - e
  keys_pad = jnp.concatenate(
      [key_indices, n_ast + (jnp.arange(pad_e, dtype=jnp.int32) % 96)])
  vi_pad = jnp.concatenate(
      [value_indices, jnp.arange(pad_e, dtype=jnp.int32) % n_cfg])

  lwin_all = _sc_winpos_phase1(keys_pad, n_pad, e_pad)
  winpos_p, vsel_p = _sc_winpos_phase2(lwin_all, vi_pad, n_pad, n_cfg)

  upd_sel = _sc_row_gather(vsel_p, new_cfg_nodes_encodings, n_pad, d,
                           chunk=160)

  wg1 = W_gate[:d]
  wg2 = W_gate[d:]
  bu = b_update.reshape(1, d)
  bg = b_gate.reshape(1, d)
  winpos2d = winpos_p.reshape(n_pad, 1)

  out = _tc_mlp(previous_ast_nodes_encodings, upd_sel, winpos2d,
                W_update, wg1, wg2, bu, bg, blk=800)
  return out


# speedup vs baseline: 10.8200x; 1.2414x over previous
import functools

import jax
import jax.numpy as jnp
from jax import lax
from jax.experimental import pallas as pl
from jax.experimental.pallas import tpu as pltpu
from jax.experimental.pallas import tpu_sc as plsc

_NC = 2
_NS = 16
_NW = _NC * _NS


def _sc_winpos_phase1(keys_pad, vals_pad, n_rows_pad, e_pad):
  per_w = e_pad // _NW
  nvec = per_w // 16
  mesh = plsc.VectorSubcoreMesh(core_axis_name="c", subcore_axis_name="s")

  slice_w = n_rows_pad // _NW

  @functools.partial(
      pl.kernel,
      out_type=jax.ShapeDtypeStruct((_NW, _NW, slice_w), jnp.int32),
      mesh=mesh,
      compiler_params=pltpu.CompilerParams(needs_layout_passes=False),
      scratch_types=[
          pltpu.VMEM((per_w,), jnp.int32),
          pltpu.VMEM((per_w,), jnp.int32),
          pltpu.VMEM((n_rows_pad,), jnp.int32),
          pltpu.SemaphoreType.DMA,
      ],
  )
  def k(keys_hbm, vals_hbm, out_hbm, keys_v, vals_v, lval, osem):
    wid = lax.axis_index("s") * _NC + lax.axis_index("c")
    base = wid * per_w
    pltpu.sync_copy(keys_hbm.at[pl.ds(base, per_w)], keys_v)
    pltpu.sync_copy(vals_hbm.at[pl.ds(base, per_w)], vals_v)

    minus1 = jnp.full((16,), -1, jnp.int32)

    def init_body(i, carry):
      for j in range(8):
        lval[pl.ds((i * 8 + j) * 16, 16)] = minus1
      return carry

    lax.fori_loop(0, n_rows_pad // (16 * 8), init_body, 0)

    lane = lax.iota(jnp.int32, 16)

    def scat_body(i, carry):
      keys = keys_v[pl.ds(i * 16, 16)]
      vals = vals_v[pl.ds(i * 16, 16)]
      for l in range(16):
        plsc.store_scatter(lval, [keys], vals, mask=lane == l)
      return carry

    lax.fori_loop(0, nvec, scat_body, 0)
    for s in range(_NW):
      pltpu.async_copy(lval.at[pl.ds(s * slice_w, slice_w)],
                       out_hbm.at[s, wid], osem)
    for s in range(_NW):
      pltpu.make_async_copy(lval.at[pl.ds(s * slice_w, slice_w)],
                            out_hbm.at[s, wid], osem).wait()

  return k(keys_pad, vals_pad)


def _sc_winpos_phase2(lval_all, n_rows_pad, n_cfg):
  per_w = n_rows_pad // _NW
  nv = per_w // 16
  mesh = plsc.VectorSubcoreMesh(core_axis_name="c", subcore_axis_name="s")

  @functools.partial(
      pl.kernel,
      out_type=(jax.ShapeDtypeStruct((n_rows_pad,), jnp.int32),
                jax.ShapeDtypeStruct((n_rows_pad,), jnp.int32)),
      mesh=mesh,
      scratch_types=[
          pltpu.VMEM((_NW, per_w), jnp.int32),
          pltpu.VMEM((per_w,), jnp.int32),
          pltpu.VMEM((per_w,), jnp.int32),
      ],
  )
  def k(lval_hbm, wp_hbm, vs_hbm, tbl_v, wp_v, vs_v):
    wid = lax.axis_index("s") * _NC + lax.axis_index("c")
    base = wid * per_w
    pltpu.sync_copy(lval_hbm.at[wid], tbl_v)

    lane = lax.iota(jnp.int32, 16)
    dummy_mask = 32767
    assert n_cfg > dummy_mask

    def max_body(i, carry):
      acc = tbl_v[0, pl.ds(i * 16, 16)]
      for t in range(1, _NW):
        cur = tbl_v[t, pl.ds(i * 16, 16)]
        acc = jnp.where(cur >= 0, cur, acc)
      wp_v[pl.ds(i * 16, 16)] = acc
      rows = base + i * 16 + lane
      vs_v[pl.ds(i * 16, 16)] = jnp.where(acc >= 0, acc, rows & dummy_mask)
      return carry

    lax.fori_loop(0, nv, max_body, 0)
    pltpu.sync_copy(wp_v, wp_hbm.at[pl.ds(base, per_w)])
    pltpu.sync_copy(vs_v, vs_hbm.at[pl.ds(base, per_w)])

  return k(lval_all)


def _sc_row_gather(vsel, cfg, n_pad, d, chunk):
  rows_per_w = n_pad // _NW
  n_chunks = rows_per_w // chunk
  mesh = plsc.VectorSubcoreMesh(core_axis_name="c", subcore_axis_name="s")

  @functools.partial(
      pl.kernel,
      out_type=jax.ShapeDtypeStruct((n_pad, d), jnp.float32),
      mesh=mesh,
      scratch_types=[
          pltpu.VMEM((rows_per_w,), jnp.int32),
          pltpu.VMEM((chunk, d), jnp.float32),
          pltpu.VMEM((chunk, d), jnp.float32),
          pltpu.SemaphoreType.DMA,
          pltpu.SemaphoreType.DMA,
      ],
  )
  def k(vsel_hbm, cfg_hbm, out_hbm, idx_v, buf0, buf1, sem0, sem1):
    wid = lax.axis_index("s") * _NC + lax.axis_index("c")
    base = wid * rows_per_w
    pltpu.sync_copy(vsel_hbm.at[pl.ds(base, rows_per_w)], idx_v)

    def gather_chunk(c, buf, sem):
      pltpu.async_copy(cfg_hbm.at[idx_v.at[pl.ds(c * chunk, chunk)]], buf, sem)

    gather_chunk(0, buf0, sem0)
    for c in range(n_chunks):
      cur, csem = (buf0, sem0) if c % 2 == 0 else (buf1, sem1)
      nxt, nsem = (buf1, sem1) if c % 2 == 0 else (buf0, sem0)
      if c + 1 < n_chunks:
        gather_chunk(c + 1, nxt, nsem)
      pltpu.make_async_copy(
          cfg_hbm.at[idx_v.at[pl.ds(c * chunk, chunk)]], cur, csem).wait()
      pltpu.sync_copy(cur, out_hbm.at[pl.ds(base + c * chunk, chunk)])

  return k(vsel, cfg)


def _tc_mlp_body(prev_ref, upd_ref, wp_ref, wu_ref, wg1_ref, wg2_ref,
                 bu_ref, bg_ref, out_ref):
  prev = prev_ref[...]
  upd = upd_ref[...]
  proj = jnp.maximum(
      jnp.dot(upd, wu_ref[...], preferred_element_type=jnp.float32)
      + bu_ref[...], 0.0)
  z = (jnp.dot(prev, wg1_ref[...], preferred_element_type=jnp.float32)
       + jnp.dot(proj, wg2_ref[...], preferred_element_type=jnp.float32)
       + bg_ref[...])
  gate = jax.nn.sigmoid(z)
  newr = gate * prev + (1.0 - gate) * proj
  out_ref[...] = jnp.where(wp_ref[...] >= 0, newr, prev)


def _tc_mlp(prev_table, upd_sel, winpos2d, wu, wg1, wg2, bu, bg, blk):
  n, d = prev_table.shape
  grid = (n // blk,)
  return pl.pallas_call(
      _tc_mlp_body,
      grid=grid,
      in_specs=[
          pl.BlockSpec((blk, d), lambda i: (i, 0)),
          pl.BlockSpec((blk, d), lambda i: (i, 0)),
          pl.BlockSpec((blk, 1), lambda i: (i, 0)),
          pl.BlockSpec((d, d), lambda i: (0, 0)),
          pl.BlockSpec((d, d), lambda i: (0, 0)),
          pl.BlockSpec((d, d), lambda i: (0, 0)),
          pl.BlockSpec((1, d), lambda i: (0, 0)),
          pl.BlockSpec((1, d), lambda i: (0, 0)),
      ],
      out_specs=pl.BlockSpec((blk, d), lambda i: (i, 0)),
      out_shape=jax.ShapeDtypeStruct((n, d), jnp.float32),
  )(prev_table, upd_sel, winpos2d, wu, wg1, wg2, bu, bg)


def kernel(previous_ast_nodes_encodings, new_cfg_nodes_encodings, key_indices,
           value_indices, W_update, b_update, W_gate, b_gate):
  n_ast, d = previous_ast_nodes_encodings.shape
  n_cfg = new_cfg_nodes_encodings.shape[0]
  e = key_indices.shape[0]

  key_indices = key_indices.astype(jnp.int32)
  value_indices = value_indices.astype(jnp.int32)

  n_pad = 102400
  e_pad = 200704

  pad_e = e_pad - e
  keys_pad = jnp.concatenate(
      [key_indices, n_ast + (jnp.arange(pad_e, dtype=jnp.int32) % 96)])
  vi_pad = jnp.concatenate(
      [value_indices, jnp.arange(pad_e, dtype=jnp.int32) % n_cfg])

  lval_all = _sc_winpos_phase1(keys_pad, vi_pad, n_pad, e_pad)
  winpos_p, vsel_p = _sc_winpos_phase2(lval_all, n_pad, n_cfg)

  upd_sel = _sc_row_gather(vsel_p, new_cfg_nodes_encodings, n_pad, d,
                           chunk=160)

  wg1 = W_gate[:d]
  wg2 = W_gate[d:]
  bu = b_update.reshape(1, d)
  bg = b_gate.reshape(1, d)
  winpos2d = winpos_p.reshape(n_pad, 1)

  out = _tc_mlp(previous_ast_nodes_encodings, upd_sel, winpos2d,
                W_update, wg1, wg2, bu, bg, blk=800)
  return out
